# baseline (speedup 1.0000x reference)
import functools

import jax
import jax.numpy as jnp
from jax import lax
from jax.experimental import pallas as pl
from jax.experimental.pallas import tpu as pltpu
from jax.experimental.pallas import tpu_sc as plsc

_ROWS = 8192
_FEAT = 256
_NC = 2
_NS = 16
_NW = _NC * _NS
_ROWS_PER_W = _ROWS // _NW


def _sc_copy(src_hbm, out_hbm, buf):
    wid = lax.axis_index("s") * _NC + lax.axis_index("c")
    base = wid * _ROWS_PER_W
    pltpu.sync_copy(
        src_hbm.at[pl.ds(base, _ROWS_PER_W)], out_hbm.at[pl.ds(base, _ROWS_PER_W)]
    )


def kernel(prototypes):
    mesh = plsc.VectorSubcoreMesh(core_axis_name="c", subcore_axis_name="s")
    k = functools.partial(
        pl.kernel,
        mesh=mesh,
        out_type=jax.ShapeDtypeStruct((_ROWS, _FEAT), jnp.float32),
        scratch_types=[pltpu.VMEM((_ROWS_PER_W, _FEAT), jnp.float32)],
    )(_sc_copy)
    return k(prototypes)


# speedup vs baseline: 41.2798x; 41.2798x over previous
import jax
import jax.numpy as jnp
from jax.experimental import pallas as pl
from jax.experimental.pallas import tpu as pltpu


_CHUNK_ROWS = (1024, 3072, 4096)


def _copy_kernel(src_ref, dst_ref, buf, in_sems, out_sems):
    offs = [0]
    for r in _CHUNK_ROWS[:-1]:
        offs.append(offs[-1] + r)
    ins, outs = [], []
    for i, (o, r) in enumerate(zip(offs, _CHUNK_ROWS)):
        c = pltpu.make_async_copy(
            src_ref.at[pl.ds(o, r)], buf.at[pl.ds(o, r)], in_sems.at[i]
        )
        c.start()
        ins.append(c)
        outs.append(
            pltpu.make_async_copy(
                buf.at[pl.ds(o, r)], dst_ref.at[pl.ds(o, r)], out_sems.at[i]
            )
        )
    for i in range(len(_CHUNK_ROWS)):
        ins[i].wait()
        outs[i].start()
    for c in outs:
        c.wait()


def kernel(prototypes):
    rows, feat = prototypes.shape
    n = len(_CHUNK_ROWS)
    return pl.pallas_call(
        _copy_kernel,
        out_shape=jax.ShapeDtypeStruct(prototypes.shape, prototypes.dtype),
        in_specs=[pl.BlockSpec(memory_space=pl.ANY)],
        out_specs=pl.BlockSpec(memory_space=pl.ANY),
        scratch_shapes=[
            pltpu.VMEM((rows, feat), prototypes.dtype),
            pltpu.SemaphoreType.DMA((n,)),
            pltpu.SemaphoreType.DMA((n,)),
        ],
    )(prototypes)
